# baseline (speedup 1.0000x reference)
import functools

import jax
import jax.numpy as jnp
from jax import lax
from jax.experimental import pallas as pl

NCOMP = 16
NCLASS = 2
D = 768
NOUT = NCOMP * NCLASS
TILE = 2048


def _select(cid_blk, logits):
    lane = lax.broadcasted_iota(jnp.int32, (TILE, NOUT), 1)
    sel = (lane // NCLASS) == cid_blk
    masked = jnp.where(sel, logits, 0.0)
    even = (lane % NCLASS) == 0
    out0 = jnp.sum(jnp.where(even, masked, 0.0), axis=1, keepdims=True)
    out1 = jnp.sum(jnp.where(even, 0.0, masked), axis=1, keepdims=True)
    return jnp.concatenate([out0, out1], axis=1)


def _mm_select_kernel(cid_ref, emb0_ref, emb1_ref, w_ref, b_ref, out_ref):
    l0 = jnp.dot(emb0_ref[...], w_ref[...], preferred_element_type=jnp.float32)
    l1 = jnp.dot(emb1_ref[...], w_ref[...], preferred_element_type=jnp.float32)
    bb = b_ref[...]
    out_ref[0:TILE, :] = _select(cid_ref[0:TILE, :], l0 + bb)
    out_ref[TILE : 2 * TILE, :] = _select(cid_ref[TILE : 2 * TILE, :], l1 + bb)


def kernel(embeddings, component_idx, W, b):
    B = embeddings.shape[0]
    cid = component_idx.astype(jnp.int32).reshape(B, 1)
    w_full = jnp.transpose(W, (1, 0, 2)).reshape(D, NOUT)
    b_full = b.reshape(1, NOUT)

    out = pl.pallas_call(
        _mm_select_kernel,
        grid=(B // (2 * TILE),),
        in_specs=[
            pl.BlockSpec((2 * TILE, 1), lambda i: (i, 0)),
            pl.BlockSpec((TILE, D), lambda i: (2 * i, 0)),
            pl.BlockSpec((TILE, D), lambda i: (2 * i + 1, 0)),
            pl.BlockSpec((D, NOUT), lambda i: (0, 0)),
            pl.BlockSpec((1, NOUT), lambda i: (0, 0)),
        ],
        out_specs=pl.BlockSpec((2 * TILE, NCLASS), lambda i: (i, 0)),
        out_shape=jax.ShapeDtypeStruct((B, NCLASS), jnp.float32),
    )(cid, embeddings, embeddings, w_full, b_full)
    return out
